# baseline (speedup 1.0000x reference)
import functools

import jax
import jax.numpy as jnp
from jax import lax
from jax.experimental import pallas as pl
from jax.experimental.pallas import tpu as pltpu
from jax.experimental.pallas import tpu_sc as plsc

D_MODEL = 768
NUM_EXPERTS = 64
D_EXPERT = 64
TOKENS = 2048
EPB = 16
GRID = NUM_EXPERTS // EPB

NUM_WORKERS = 16
TOK_PER_BLK = TOKENS // NUM_WORKERS
LANES = 16


def _router_kernel(x_ref, wg_ref, probsT_ref, pm_ref):
    x = x_ref[...]
    logits = jnp.dot(x, wg_ref[...], preferred_element_type=jnp.float32)
    m = jnp.max(logits, axis=-1, keepdims=True)
    ex = jnp.exp(logits - m)
    probs = ex / jnp.sum(ex, axis=-1, keepdims=True)
    probsT_ref[...] = probs.T
    pm_ref[...] = jnp.sum(probs, axis=0, keepdims=True)


NGRP = TOK_PER_BLK // LANES


def _sc_top2_body(probsT_hbm, cwT_hbm, probs_v, cw_v):
    cid = lax.axis_index("c")
    sid = lax.axis_index("s")
    wid = cid * 8 + sid

    @pl.when(sid < 8)
    def _work():
        base = wid * TOK_PER_BLK
        pltpu.sync_copy(probsT_hbm.at[:, pl.ds(base, TOK_PER_BLK)], probs_v)
        neg = jnp.full((LANES,), -jnp.inf, jnp.float32)
        izero = jnp.zeros((LANES,), jnp.int32)

        def scan_expert(e, carry):
            v1, i1, v2, i2 = carry
            ev = jnp.broadcast_to(e, (LANES,))
            nv1, ni1, nv2, ni2 = [], [], [], []
            for j in range(NGRP):
                p = probs_v[e, pl.ds(j * LANES, LANES)]
                gt1 = p > v1[j]
                gt2 = p > v2[j]
                nv2.append(jnp.where(gt1, v1[j], jnp.where(gt2, p, v2[j])))
                ni2.append(jnp.where(gt1, i1[j], jnp.where(gt2, ev, i2[j])))
                nv1.append(jnp.where(gt1, p, v1[j]))
                ni1.append(jnp.where(gt1, ev, i1[j]))
            return tuple(nv1), tuple(ni1), tuple(nv2), tuple(ni2)

        init = (
            (neg,) * NGRP, (izero,) * NGRP, (neg,) * NGRP, (izero,) * NGRP,
        )
        v1, i1, v2, i2 = lax.fori_loop(0, NUM_EXPERTS, scan_expert, init)
        g1 = []
        g2 = []
        for j in range(NGRP):
            sv = v1[j] + v2[j]
            g1.append(v1[j] / sv + 2.0)
            g2.append(v2[j] / sv + 2.0)
        zero = jnp.zeros((LANES,), jnp.float32)

        def write_expert(e, carry):
            ev = jnp.broadcast_to(e, (LANES,))
            for j in range(NGRP):
                row = (jnp.where(i1[j] == ev, g1[j], zero)
                       + jnp.where(i2[j] == ev, g2[j], zero))
                cw_v[e, pl.ds(j * LANES, LANES)] = row
            return carry

        lax.fori_loop(0, NUM_EXPERTS, write_expert, 0)
        pltpu.sync_copy(cw_v, cwT_hbm.at[:, pl.ds(base, TOK_PER_BLK)])


def _ffn_kernel(x_ref, cwm_ref, pm_ref, w1_ref, w2_ref, out_ref, aux_ref,
                cw_ref):
    g = pl.program_id(0)

    @pl.when(g == 0)
    def _decode():
        cwmT = cwm_ref[...]
        mask = (cwmT > 1.5).astype(jnp.float32)
        cw_ref[...] = (cwmT - 2.0 * mask).T
        rc = jnp.sum(mask, axis=1, keepdims=True)
        aux = (NUM_EXPERTS / (TOKENS * TOKENS)) * jnp.dot(
            pm_ref[...], rc, preferred_element_type=jnp.float32)
        aux_ref[...] = aux

    x = x_ref[...]
    h = jnp.dot(x, w1_ref[...], preferred_element_type=jnp.float32)
    h = 0.5 * h * (1.0 + jax.lax.erf(h * 0.7071067811865476))
    er = jax.lax.broadcasted_iota(jnp.int32, (NUM_EXPERTS, EPB * D_EXPERT), 0)
    ec = jax.lax.broadcasted_iota(jnp.int32, (NUM_EXPERTS, EPB * D_EXPERT), 1)
    sel = (er == g * EPB + ec // D_EXPERT).astype(jnp.float32)
    scale = jnp.dot(cw_ref[...], sel, preferred_element_type=jnp.float32)
    h = h * scale
    contrib = jnp.dot(h, w2_ref[...], preferred_element_type=jnp.float32)

    @pl.when(g == 0)
    def _():
        out_ref[...] = contrib

    @pl.when(g > 0)
    def _():
        out_ref[...] += contrib


def kernel(x, W_gate, expert_w1, expert_w2):
    x2d = x.reshape(TOKENS, D_MODEL)
    w1_all = expert_w1.transpose(1, 0, 2).reshape(D_MODEL, NUM_EXPERTS * D_EXPERT)
    w2_all = expert_w2.reshape(NUM_EXPERTS * D_EXPERT, D_MODEL)

    probsT, pm = pl.pallas_call(
        _router_kernel,
        out_shape=(
            jax.ShapeDtypeStruct((NUM_EXPERTS, TOKENS), jnp.float32),
            jax.ShapeDtypeStruct((1, NUM_EXPERTS), jnp.float32),
        ),
    )(x2d, W_gate)

    sc_top2 = functools.partial(
        pl.kernel,
        mesh=plsc.VectorSubcoreMesh(core_axis_name="c", subcore_axis_name="s"),
        out_type=jax.ShapeDtypeStruct((NUM_EXPERTS, TOKENS), jnp.float32),
        scratch_types=[
            pltpu.VMEM((NUM_EXPERTS, TOK_PER_BLK), jnp.float32),
            pltpu.VMEM((NUM_EXPERTS, TOK_PER_BLK), jnp.float32),
        ],
    )(_sc_top2_body)
    cwmT = sc_top2(probsT)

    out2d, aux = pl.pallas_call(
        _ffn_kernel,
        grid=(GRID,),
        in_specs=[
            pl.BlockSpec((TOKENS, D_MODEL), lambda g: (0, 0)),
            pl.BlockSpec((NUM_EXPERTS, TOKENS), lambda g: (0, 0)),
            pl.BlockSpec((1, NUM_EXPERTS), lambda g: (0, 0)),
            pl.BlockSpec((D_MODEL, EPB * D_EXPERT), lambda g: (0, g)),
            pl.BlockSpec((EPB * D_EXPERT, D_MODEL), lambda g: (g, 0)),
        ],
        out_specs=(
            pl.BlockSpec((TOKENS, D_MODEL), lambda g: (0, 0)),
            pl.BlockSpec((1, 1), lambda g: (0, 0)),
        ),
        out_shape=(
            jax.ShapeDtypeStruct((TOKENS, D_MODEL), jnp.float32),
            jax.ShapeDtypeStruct((1, 1), jnp.float32),
        ),
        scratch_shapes=[pltpu.VMEM((TOKENS, NUM_EXPERTS), jnp.float32)],
    )(x2d, cwmT, pm, w1_all, w2_all)

    return out2d.reshape(x.shape), aux[0, 0]
